# baseline (speedup 1.0000x reference)
import functools

import jax
import jax.numpy as jnp
from jax import lax
from jax.experimental import pallas as pl
from jax.experimental.pallas import tpu as pltpu
from jax.experimental.pallas import tpu_sc as plsc

N, E, D = 10000, 320000, 128
L = 16
NT = 16
CHUNK = 64
NCHT = E // CHUNK
BLK = 8
NBLK = NCHT // BLK
ACC_ROWS = 10240
SEG = 640
CP = 64


def _graph_body(t, f_hbm, s_hbm, d_hbm, agg_hbm,
                src_stage, dst_stage, gbuf, zidx, acc,
                sem0, sem1, ssem0, ssem1):
    def zrow(i, _):
        def zcol(j, _):
            gbuf[0, i, pl.ds(j * L, L)] = jnp.zeros((L,), jnp.float32)
            return 0
        lax.fori_loop(0, D // L, zcol, 0)
        return 0
    lax.fori_loop(0, CHUNK, zrow, 0)

    iota = lax.iota(jnp.int32, L)

    def zirow(r, _):
        for j in range(CHUNK // L):
            zidx[r, pl.ds(j * L, L)] = (
                jnp.zeros((L,), jnp.int32) + t * SEG + r * CP + j * L + iota)
        return 0
    lax.fori_loop(0, SEG // CP, zirow, 0)

    for k in range(SEG // CP):
        pltpu.sync_copy(gbuf.at[0], acc.at[zidx.at[k]])

    plsc.subcore_barrier()

    nblk = 39 + jnp.where(t < 1, 1, 0)
    base_blk = t * 39 + jnp.minimum(t, 1)

    def block(blk, _):
        row = (base_blk + blk) * BLK
        pltpu.sync_copy(s_hbm.at[pl.ds(row, BLK)], src_stage)
        pltpu.sync_copy(d_hbm.at[pl.ds(row, BLK)], dst_stage)

        sems = (sem0, sem1)
        ssems = (ssem0, ssem1)
        pltpu.async_copy(f_hbm.at[src_stage.at[0]], gbuf.at[0], sems[0])
        sc_handles = [None, None]
        for j in range(BLK):
            pltpu.make_async_copy(
                f_hbm.at[src_stage.at[j]], gbuf.at[j % 2], sems[j % 2]).wait()
            if j + 1 < BLK:
                if sc_handles[(j + 1) % 2] is not None:
                    sc_handles[(j + 1) % 2].wait()
                pltpu.async_copy(f_hbm.at[src_stage.at[j + 1]],
                                 gbuf.at[(j + 1) % 2], sems[(j + 1) % 2])
            sc_handles[j % 2] = pltpu.async_copy(
                gbuf.at[j % 2], acc.at[dst_stage.at[j]], ssems[j % 2],
                add=True)
        for h in sc_handles:
            if h is not None:
                h.wait()
        return 0
    lax.fori_loop(0, nblk, block, 0)

    plsc.subcore_barrier()

    for k in range(SEG // CP):
        pltpu.sync_copy(acc.at[zidx.at[k]], gbuf.at[0])
        pltpu.sync_copy(gbuf.at[0], agg_hbm.at[pl.ds(t * SEG + k * CP, CP)])


@functools.partial(
    pl.kernel,
    out_type=(
        jax.ShapeDtypeStruct((ACC_ROWS, D), jnp.float32),
        jax.ShapeDtypeStruct((ACC_ROWS, D), jnp.float32),
    ),
    mesh=plsc.VectorSubcoreMesh(core_axis_name="c", subcore_axis_name="s"),
    scratch_types=[
        pltpu.VMEM((BLK, CHUNK), jnp.int32),
        pltpu.VMEM((BLK, CHUNK), jnp.int32),
        pltpu.VMEM((2, CHUNK, D), jnp.float32),
        pltpu.VMEM((SEG // CP, CHUNK), jnp.int32),
        pltpu.VMEM_SHARED((ACC_ROWS, D), jnp.float32),
        pltpu.SemaphoreType.DMA,
        pltpu.SemaphoreType.DMA,
        pltpu.SemaphoreType.DMA,
        pltpu.SemaphoreType.DMA,
    ],
)
def _sc_aggregate(f0, s0, d0, f1, s1, d1, agg0, agg1,
                  src_stage, dst_stage, gbuf, zidx, acc,
                  sem0, sem1, ssem0, ssem1):
    c = lax.axis_index("c")
    t = lax.axis_index("s")

    @pl.when(c == 0)
    def _():
        _graph_body(t, f0, s0, d0, agg0,
                    src_stage, dst_stage, gbuf, zidx, acc,
                    sem0, sem1, ssem0, ssem1)

    @pl.when(c == 1)
    def _():
        _graph_body(t, f1, s1, d1, agg1,
                    src_stage, dst_stage, gbuf, zidx, acc,
                    sem0, sem1, ssem0, ssem1)


def _deg_body(t, d_hbm, deg_hbm, dst_stage, ones_buf, zdeg_buf, zidx,
              deg2d, sem):
    def zrow(i, _):
        def zcol(j, _):
            zdeg_buf[i, pl.ds(j * L, L)] = jnp.zeros((L,), jnp.float32)
            ones_buf[i, pl.ds(j * L, L)] = jnp.ones((L,), jnp.float32)
            return 0
        lax.fori_loop(0, D // L, zcol, 0)
        return 0
    lax.fori_loop(0, CHUNK, zrow, 0)

    iota = lax.iota(jnp.int32, L)

    def zirow(r, _):
        for j in range(CHUNK // L):
            zidx[r, pl.ds(j * L, L)] = (
                jnp.zeros((L,), jnp.int32) + t * SEG + r * CP + j * L + iota)
        return 0
    lax.fori_loop(0, SEG // CP, zirow, 0)

    for k in range(SEG // CP):
        pltpu.sync_copy(zdeg_buf, deg2d.at[zidx.at[k]])

    plsc.subcore_barrier()

    nblk = 39 + jnp.where(t < 1, 1, 0)
    base_blk = t * 39 + jnp.minimum(t, 1)

    def block(blk, _):
        row = (base_blk + blk) * BLK
        pltpu.sync_copy(d_hbm.at[pl.ds(row, BLK)], dst_stage)
        handles = [
            pltpu.async_copy(ones_buf, deg2d.at[dst_stage.at[j]], sem,
                             add=True)
            for j in range(BLK)
        ]
        for h in handles:
            h.wait()
        return 0
    lax.fori_loop(0, nblk, block, 0)

    plsc.subcore_barrier()

    for k in range(SEG // CP):
        pltpu.sync_copy(deg2d.at[zidx.at[k]], zdeg_buf)
        pltpu.sync_copy(zdeg_buf, deg_hbm.at[pl.ds(t * SEG + k * CP, CP)])


@functools.partial(
    pl.kernel,
    out_type=(
        jax.ShapeDtypeStruct((ACC_ROWS, D), jnp.float32),
        jax.ShapeDtypeStruct((ACC_ROWS, D), jnp.float32),
    ),
    mesh=plsc.VectorSubcoreMesh(core_axis_name="c", subcore_axis_name="s"),
    scratch_types=[
        pltpu.VMEM((BLK, CHUNK), jnp.int32),
        pltpu.VMEM((CHUNK, D), jnp.float32),
        pltpu.VMEM((CHUNK, D), jnp.float32),
        pltpu.VMEM((SEG // CP, CHUNK), jnp.int32),
        pltpu.VMEM_SHARED((ACC_ROWS, D), jnp.float32),
        pltpu.SemaphoreType.DMA,
    ],
)
def _sc_degree(d0, d1, deg0, deg1,
               dst_stage, ones_buf, zdeg_buf, zidx, deg2d, sem):
    c = lax.axis_index("c")
    t = lax.axis_index("s")

    @pl.when(c == 0)
    def _():
        _deg_body(t, d0, deg0, dst_stage, ones_buf, zdeg_buf, zidx,
                  deg2d, sem)

    @pl.when(c == 1)
    def _():
        _deg_body(t, d1, deg1, dst_stage, ones_buf, zdeg_buf, zidx,
                  deg2d, sem)


def _tc_body(x_ref, agg_ref, deg_ref, ws_ref, wn_ref, b_ref, o_ref):
    h = agg_ref[...] / jnp.maximum(deg_ref[:, 0:1], 1.0)
    o_ref[...] = (
        jnp.dot(x_ref[...], ws_ref[...], preferred_element_type=jnp.float32)
        + jnp.dot(h, wn_ref[...], preferred_element_type=jnp.float32)
        + b_ref[...])


BN = 2000


def _tc_combine(x, agg, deg, W_self, W_neigh, b2):
    return pl.pallas_call(
        _tc_body,
        grid=(N // BN,),
        in_specs=[
            pl.BlockSpec((BN, D), lambda i: (i, 0)),
            pl.BlockSpec((BN, D), lambda i: (i, 0)),
            pl.BlockSpec((BN, D), lambda i: (i, 0)),
            pl.BlockSpec((D, D), lambda i: (0, 0)),
            pl.BlockSpec((D, D), lambda i: (0, 0)),
            pl.BlockSpec((1, D), lambda i: (0, 0)),
        ],
        out_specs=pl.BlockSpec((BN, D), lambda i: (i, 0)),
        out_shape=jax.ShapeDtypeStruct((N, D), jnp.float32),
    )(x, agg, deg, W_self, W_neigh, b2)


def kernel(feats0, feats1, edge_index0, edge_index1, W_self, W_neigh, b):
    s0 = edge_index0[0].reshape(NCHT, CHUNK)
    d0 = edge_index0[1].reshape(NCHT, CHUNK)
    s1 = edge_index1[0].reshape(NCHT, CHUNK)
    d1 = edge_index1[1].reshape(NCHT, CHUNK)
    agg0, agg1 = _sc_aggregate(feats0, s0, d0, feats1, s1, d1)
    deg0, deg1 = _sc_degree(d0, d1)
    b2 = b.reshape(1, D)
    out0 = _tc_combine(feats0, agg0, deg0, W_self, W_neigh, b2)
    out1 = _tc_combine(feats1, agg1, deg1, W_self, W_neigh, b2)
    return out0, out1


# speedup vs baseline: 1.2666x; 1.2666x over previous
import functools

import jax
import jax.numpy as jnp
from jax import lax
from jax.experimental import pallas as pl
from jax.experimental.pallas import tpu as pltpu
from jax.experimental.pallas import tpu_sc as plsc

N, E, D = 10000, 320000, 128
L = 16
NT = 16
CHUNK = 128
NCHT = E // CHUNK
PADR = 2504
BLK = 8
NBLK = PADR // BLK
ACC_ROWS = 10240
SEG = 640
CP = 128


def _graph_body(t, f_hbm, s_hbm, d_hbm, agg_hbm,
                src_stage, dst_stage, gbuf, zidx, acc,
                sem0, sem1, ssem0, ssem1):
    def zrow(i, _):
        def zcol(j, _):
            gbuf[0, i, pl.ds(j * L, L)] = jnp.zeros((L,), jnp.float32)
            return 0
        lax.fori_loop(0, D // L, zcol, 0)
        return 0
    lax.fori_loop(0, CHUNK, zrow, 0)

    iota = lax.iota(jnp.int32, L)

    def zirow(r, _):
        for j in range(CHUNK // L):
            zidx[r, pl.ds(j * L, L)] = (
                jnp.zeros((L,), jnp.int32) + t * SEG + r * CP + j * L + iota)
        return 0
    lax.fori_loop(0, SEG // CP, zirow, 0)

    for k in range(SEG // CP):
        pltpu.sync_copy(gbuf.at[0], acc.at[zidx.at[k]])

    plsc.subcore_barrier()

    nblk = 19 + jnp.where(t < 9, 1, 0)
    base_blk = t * 19 + jnp.minimum(t, 9)

    def block(blk, _):
        row = (base_blk + blk) * BLK
        pltpu.sync_copy(s_hbm.at[pl.ds(row, BLK)], src_stage)
        pltpu.sync_copy(d_hbm.at[pl.ds(row, BLK)], dst_stage)

        sems = (sem0, sem1)
        ssems = (ssem0, ssem1)
        pltpu.async_copy(f_hbm.at[src_stage.at[0]], gbuf.at[0], sems[0])
        sc_handles = [None, None]
        for j in range(BLK):
            pltpu.make_async_copy(
                f_hbm.at[src_stage.at[j]], gbuf.at[j % 2], sems[j % 2]).wait()
            if j + 1 < BLK:
                if sc_handles[(j + 1) % 2] is not None:
                    sc_handles[(j + 1) % 2].wait()
                pltpu.async_copy(f_hbm.at[src_stage.at[j + 1]],
                                 gbuf.at[(j + 1) % 2], sems[(j + 1) % 2])
            sc_handles[j % 2] = pltpu.async_copy(
                gbuf.at[j % 2], acc.at[dst_stage.at[j]], ssems[j % 2],
                add=True)
        for h in sc_handles:
            if h is not None:
                h.wait()
        return 0
    lax.fori_loop(0, nblk, block, 0)

    plsc.subcore_barrier()

    for k in range(SEG // CP):
        pltpu.sync_copy(acc.at[zidx.at[k]], gbuf.at[0])
        pltpu.sync_copy(gbuf.at[0], agg_hbm.at[pl.ds(t * SEG + k * CP, CP)])


@functools.partial(
    pl.kernel,
    out_type=(
        jax.ShapeDtypeStruct((ACC_ROWS, D), jnp.float32),
        jax.ShapeDtypeStruct((ACC_ROWS, D), jnp.float32),
    ),
    mesh=plsc.VectorSubcoreMesh(core_axis_name="c", subcore_axis_name="s"),
    scratch_types=[
        pltpu.VMEM((BLK, CHUNK), jnp.int32),
        pltpu.VMEM((BLK, CHUNK), jnp.int32),
        pltpu.VMEM((2, CHUNK, D), jnp.float32),
        pltpu.VMEM((SEG // CP, CHUNK), jnp.int32),
        pltpu.VMEM_SHARED((ACC_ROWS, D), jnp.float32),
        pltpu.SemaphoreType.DMA,
        pltpu.SemaphoreType.DMA,
        pltpu.SemaphoreType.DMA,
        pltpu.SemaphoreType.DMA,
    ],
)
def _sc_aggregate(f0, s0, d0, f1, s1, d1, agg0, agg1,
                  src_stage, dst_stage, gbuf, zidx, acc,
                  sem0, sem1, ssem0, ssem1):
    c = lax.axis_index("c")
    t = lax.axis_index("s")

    @pl.when(c == 0)
    def _():
        _graph_body(t, f0, s0, d0, agg0,
                    src_stage, dst_stage, gbuf, zidx, acc,
                    sem0, sem1, ssem0, ssem1)

    @pl.when(c == 1)
    def _():
        _graph_body(t, f1, s1, d1, agg1,
                    src_stage, dst_stage, gbuf, zidx, acc,
                    sem0, sem1, ssem0, ssem1)


def _deg_body(t, d_hbm, deg_hbm, dst_stage, ones_buf, zdeg_buf, zidx,
              deg2d, sem):
    def zrow(i, _):
        def zcol(j, _):
            zdeg_buf[i, pl.ds(j * L, L)] = jnp.zeros((L,), jnp.float32)
            ones_buf[i, pl.ds(j * L, L)] = jnp.ones((L,), jnp.float32)
            return 0
        lax.fori_loop(0, D // L, zcol, 0)
        return 0
    lax.fori_loop(0, CHUNK, zrow, 0)

    iota = lax.iota(jnp.int32, L)

    def zirow(r, _):
        for j in range(CHUNK // L):
            zidx[r, pl.ds(j * L, L)] = (
                jnp.zeros((L,), jnp.int32) + t * SEG + r * CP + j * L + iota)
        return 0
    lax.fori_loop(0, SEG // CP, zirow, 0)

    for k in range(SEG // CP):
        pltpu.sync_copy(zdeg_buf, deg2d.at[zidx.at[k]])

    plsc.subcore_barrier()

    nblk = 19 + jnp.where(t < 9, 1, 0)
    base_blk = t * 19 + jnp.minimum(t, 9)

    def block(blk, _):
        row = (base_blk + blk) * BLK
        pltpu.sync_copy(d_hbm.at[pl.ds(row, BLK)], dst_stage)
        handles = [
            pltpu.async_copy(ones_buf, deg2d.at[dst_stage.at[j]], sem,
                             add=True)
            for j in range(BLK)
        ]
        for h in handles:
            h.wait()
        return 0
    lax.fori_loop(0, nblk, block, 0)

    plsc.subcore_barrier()

    for k in range(SEG // CP):
        pltpu.sync_copy(deg2d.at[zidx.at[k]], zdeg_buf)
        pltpu.sync_copy(zdeg_buf, deg_hbm.at[pl.ds(t * SEG + k * CP, CP)])


@functools.partial(
    pl.kernel,
    out_type=(
        jax.ShapeDtypeStruct((ACC_ROWS, D), jnp.float32),
        jax.ShapeDtypeStruct((ACC_ROWS, D), jnp.float32),
    ),
    mesh=plsc.VectorSubcoreMesh(core_axis_name="c", subcore_axis_name="s"),
    scratch_types=[
        pltpu.VMEM((BLK, CHUNK), jnp.int32),
        pltpu.VMEM((CHUNK, D), jnp.float32),
        pltpu.VMEM((CHUNK, D), jnp.float32),
        pltpu.VMEM((SEG // CP, CHUNK), jnp.int32),
        pltpu.VMEM_SHARED((ACC_ROWS, D), jnp.float32),
        pltpu.SemaphoreType.DMA,
    ],
)
def _sc_degree(d0, d1, deg0, deg1,
               dst_stage, ones_buf, zdeg_buf, zidx, deg2d, sem):
    c = lax.axis_index("c")
    t = lax.axis_index("s")

    @pl.when(c == 0)
    def _():
        _deg_body(t, d0, deg0, dst_stage, ones_buf, zdeg_buf, zidx,
                  deg2d, sem)

    @pl.when(c == 1)
    def _():
        _deg_body(t, d1, deg1, dst_stage, ones_buf, zdeg_buf, zidx,
                  deg2d, sem)


def _tc_body(x_ref, agg_ref, deg_ref, ws_ref, wn_ref, b_ref, o_ref):
    h = agg_ref[...] / jnp.maximum(deg_ref[:, 0:1], 1.0)
    o_ref[...] = (
        jnp.dot(x_ref[...], ws_ref[...], preferred_element_type=jnp.float32)
        + jnp.dot(h, wn_ref[...], preferred_element_type=jnp.float32)
        + b_ref[...])


BN = 2000


def _tc_combine(x, agg, deg, W_self, W_neigh, b2):
    return pl.pallas_call(
        _tc_body,
        grid=(N // BN,),
        in_specs=[
            pl.BlockSpec((BN, D), lambda i: (i, 0)),
            pl.BlockSpec((BN, D), lambda i: (i, 0)),
            pl.BlockSpec((BN, D), lambda i: (i, 0)),
            pl.BlockSpec((D, D), lambda i: (0, 0)),
            pl.BlockSpec((D, D), lambda i: (0, 0)),
            pl.BlockSpec((1, D), lambda i: (0, 0)),
        ],
        out_specs=pl.BlockSpec((BN, D), lambda i: (i, 0)),
        out_shape=jax.ShapeDtypeStruct((N, D), jnp.float32),
    )(x, agg, deg, W_self, W_neigh, b2)


def _prep_edges(ei):
    src = jnp.pad(ei[0].reshape(NCHT, CHUNK), ((0, PADR - NCHT), (0, 0)))
    dst = jnp.pad(ei[1].reshape(NCHT, CHUNK), ((0, PADR - NCHT), (0, 0)),
                  constant_values=N)
    return src, dst


def kernel(feats0, feats1, edge_index0, edge_index1, W_self, W_neigh, b):
    s0, d0 = _prep_edges(edge_index0)
    s1, d1 = _prep_edges(edge_index1)
    agg0, agg1 = _sc_aggregate(feats0, s0, d0, feats1, s1, d1)
    deg0, deg1 = _sc_degree(d0, d1)
    b2 = b.reshape(1, D)
    out0 = _tc_combine(feats0, agg0, deg0, W_self, W_neigh, b2)
    out1 = _tc_combine(feats1, agg1, deg1, W_self, W_neigh, b2)
    return out0, out1


# speedup vs baseline: 1.2795x; 1.0102x over previous
import functools

import jax
import jax.numpy as jnp
from jax import lax
from jax.experimental import pallas as pl
from jax.experimental.pallas import tpu as pltpu
from jax.experimental.pallas import tpu_sc as plsc

N, E, D = 10000, 320000, 128
L = 16
NT = 16
CHUNK = 128
NCHT = E // CHUNK
PADR = 2504
BLK = 8
NBLK = PADR // BLK
ACC_ROWS = 10240
SEG = 640
CP = 128


def _graph_body(t, f_hbm, s_hbm, d_hbm, agg_hbm, deg_hbm,
                src_stage, dst_stage, gbuf, zidx, acc,
                sem0, sem1, ssem0, ssem1):
    def zrow(i, _):
        def zcol(j, _):
            gbuf[0, i, pl.ds(j * L, L)] = jnp.zeros((L,), jnp.float32)
            return 0
        lax.fori_loop(0, D // L, zcol, 0)
        return 0
    lax.fori_loop(0, CHUNK, zrow, 0)

    iota = lax.iota(jnp.int32, L)

    def zirow(r, _):
        for j in range(CHUNK // L):
            zidx[r, pl.ds(j * L, L)] = (
                jnp.zeros((L,), jnp.int32) + t * SEG + r * CP + j * L + iota)
        return 0
    lax.fori_loop(0, SEG // CP, zirow, 0)

    for k in range(SEG // CP):
        pltpu.sync_copy(gbuf.at[0], acc.at[zidx.at[k]])

    plsc.subcore_barrier()

    nblk = 19 + jnp.where(t < 9, 1, 0)
    base_blk = t * 19 + jnp.minimum(t, 9)

    def block(blk, _):
        row = (base_blk + blk) * BLK
        pltpu.sync_copy(s_hbm.at[pl.ds(row, BLK)], src_stage)
        pltpu.sync_copy(d_hbm.at[pl.ds(row, BLK)], dst_stage)

        sems = (sem0, sem1)
        ssems = (ssem0, ssem1)
        pltpu.async_copy(f_hbm.at[src_stage.at[0]], gbuf.at[0], sems[0])
        sc_handles = [None, None]
        for j in range(BLK):
            pltpu.make_async_copy(
                f_hbm.at[src_stage.at[j]], gbuf.at[j % 2], sems[j % 2]).wait()
            if j + 1 < BLK:
                if sc_handles[(j + 1) % 2] is not None:
                    sc_handles[(j + 1) % 2].wait()
                pltpu.async_copy(f_hbm.at[src_stage.at[j + 1]],
                                 gbuf.at[(j + 1) % 2], sems[(j + 1) % 2])
            sc_handles[j % 2] = pltpu.async_copy(
                gbuf.at[j % 2], acc.at[dst_stage.at[j]], ssems[j % 2],
                add=True)
        for h in sc_handles:
            if h is not None:
                h.wait()
        return 0
    lax.fori_loop(0, nblk, block, 0)

    plsc.subcore_barrier()

    for k in range(SEG // CP):
        pltpu.sync_copy(acc.at[zidx.at[k]], gbuf.at[0])
        pltpu.sync_copy(gbuf.at[0], agg_hbm.at[pl.ds(t * SEG + k * CP, CP)])

    plsc.subcore_barrier()

    def zfill(i2, _):
        def zc(j, _):
            gbuf[0, i2, pl.ds(j * L, L)] = jnp.zeros((L,), jnp.float32)
            gbuf[1, i2, pl.ds(j * L, L)] = jnp.ones((L,), jnp.float32)
            return 0
        lax.fori_loop(0, D // L, zc, 0)
        return 0
    lax.fori_loop(0, CHUNK, zfill, 0)

    for k in range(SEG // CP):
        pltpu.sync_copy(gbuf.at[0], acc.at[zidx.at[k]])

    plsc.subcore_barrier()

    def dblock(blk, _):
        row = (base_blk + blk) * BLK
        pltpu.sync_copy(d_hbm.at[pl.ds(row, BLK)], dst_stage)
        handles = [
            pltpu.async_copy(gbuf.at[1], acc.at[dst_stage.at[j]], ssem0,
                             add=True)
            for j in range(BLK)
        ]
        for h in handles:
            h.wait()
        return 0
    lax.fori_loop(0, nblk, dblock, 0)

    plsc.subcore_barrier()

    for k in range(SEG // CP):
        pltpu.sync_copy(acc.at[zidx.at[k]], gbuf.at[0])
        pltpu.sync_copy(gbuf.at[0], deg_hbm.at[pl.ds(t * SEG + k * CP, CP)])


@functools.partial(
    pl.kernel,
    out_type=(
        jax.ShapeDtypeStruct((ACC_ROWS, D), jnp.float32),
        jax.ShapeDtypeStruct((ACC_ROWS, D), jnp.float32),
        jax.ShapeDtypeStruct((ACC_ROWS, D), jnp.float32),
        jax.ShapeDtypeStruct((ACC_ROWS, D), jnp.float32),
    ),
    mesh=plsc.VectorSubcoreMesh(core_axis_name="c", subcore_axis_name="s"),
    scratch_types=[
        pltpu.VMEM((BLK, CHUNK), jnp.int32),
        pltpu.VMEM((BLK, CHUNK), jnp.int32),
        pltpu.VMEM((2, CHUNK, D), jnp.float32),
        pltpu.VMEM((SEG // CP, CHUNK), jnp.int32),
        pltpu.VMEM_SHARED((ACC_ROWS, D), jnp.float32),
        pltpu.SemaphoreType.DMA,
        pltpu.SemaphoreType.DMA,
        pltpu.SemaphoreType.DMA,
        pltpu.SemaphoreType.DMA,
    ],
)
def _sc_aggregate(f0, s0, d0, f1, s1, d1, agg0, agg1, deg0, deg1,
                  src_stage, dst_stage, gbuf, zidx, acc,
                  sem0, sem1, ssem0, ssem1):
    c = lax.axis_index("c")
    t = lax.axis_index("s")

    @pl.when(c == 0)
    def _():
        _graph_body(t, f0, s0, d0, agg0, deg0,
                    src_stage, dst_stage, gbuf, zidx, acc,
                    sem0, sem1, ssem0, ssem1)

    @pl.when(c == 1)
    def _():
        _graph_body(t, f1, s1, d1, agg1, deg1,
                    src_stage, dst_stage, gbuf, zidx, acc,
                    sem0, sem1, ssem0, ssem1)


def _tc_body(x_ref, agg_ref, deg_ref, ws_ref, wn_ref, b_ref, o_ref):
    h = agg_ref[...] / jnp.maximum(deg_ref[:, 0:1], 1.0)
    o_ref[...] = (
        jnp.dot(x_ref[...], ws_ref[...], preferred_element_type=jnp.float32)
        + jnp.dot(h, wn_ref[...], preferred_element_type=jnp.float32)
        + b_ref[...])


BN = 2000


def _tc_combine(x, agg, deg, W_self, W_neigh, b2):
    return pl.pallas_call(
        _tc_body,
        grid=(N // BN,),
        in_specs=[
            pl.BlockSpec((BN, D), lambda i: (i, 0)),
            pl.BlockSpec((BN, D), lambda i: (i, 0)),
            pl.BlockSpec((BN, D), lambda i: (i, 0)),
            pl.BlockSpec((D, D), lambda i: (0, 0)),
            pl.BlockSpec((D, D), lambda i: (0, 0)),
            pl.BlockSpec((1, D), lambda i: (0, 0)),
        ],
        out_specs=pl.BlockSpec((BN, D), lambda i: (i, 0)),
        out_shape=jax.ShapeDtypeStruct((N, D), jnp.float32),
    )(x, agg, deg, W_self, W_neigh, b2)


def _prep_edges(ei):
    src = jnp.pad(ei[0].reshape(NCHT, CHUNK), ((0, PADR - NCHT), (0, 0)))
    dst = jnp.pad(ei[1].reshape(NCHT, CHUNK), ((0, PADR - NCHT), (0, 0)),
                  constant_values=N)
    return src, dst


def kernel(feats0, feats1, edge_index0, edge_index1, W_self, W_neigh, b):
    s0, d0 = _prep_edges(edge_index0)
    s1, d1 = _prep_edges(edge_index1)
    agg0, agg1, deg0, deg1 = _sc_aggregate(feats0, s0, d0, feats1, s1, d1)
    b2 = b.reshape(1, D)
    out0 = _tc_combine(feats0, agg0, deg0, W_self, W_neigh, b2)
    out1 = _tc_combine(feats1, agg1, deg1, W_self, W_neigh, b2)
    return out0, out1
